# baseline (speedup 1.0000x reference)
import jax
import jax.numpy as jnp
from jax import lax
from jax.experimental import pallas as pl
from jax.experimental.pallas import tpu as pltpu
from jax.experimental.pallas import tpu_sc as plsc

DEPTH = 1000
N_ROWS = 4096
N_COLS = 20
TOT = N_ROWS * N_COLS

NUM_CORES = 2
NUM_SUBCORES = 16
NW = NUM_CORES * NUM_SUBCORES
ROWS_PER_W = N_ROWS // NW
QPW = ROWS_PER_W * N_COLS

CROWS = 2
CQ = CROWS * N_COLS
NCHUNK = ROWS_PER_W // CROWS
NBUF = 2
NGROUP = NCHUNK // NBUF
NT = (CQ + 15) // 16


def _scatter_stateful(refs):
    idx_ref, zc_ref, out_ref = refs
    mesh = plsc.VectorSubcoreMesh(core_axis_name="c", subcore_axis_name="s")

    @pl.core_map(
        mesh,
        compiler_params=pltpu.CompilerParams(
            use_tc_tiling_on_sc=False, needs_layout_passes=False
        ),
    )
    def _():
        def scoped(idx_v, zbuf0, zbuf1, sem0, sem1):
            zbufs = (zbuf0, zbuf1)
            sems = (sem0, sem1)
            c = lax.axis_index("c")
            s = lax.axis_index("s")
            wid = s * NUM_CORES + c
            base_q = wid * QPW
            base_n = wid * ROWS_PER_W
            pltpu.sync_copy(idx_ref.at[pl.ds(base_q, QPW + 16)], idx_v)
            pltpu.make_async_copy(zc_ref, zbuf0, sem0).start()
            pltpu.make_async_copy(zc_ref, zbuf1, sem1).start()

            ones16 = jnp.full((16,), 1.0, jnp.float32)
            zeros16 = jnp.zeros((16,), jnp.float32)
            iota16 = lax.iota(jnp.int32, 16)

            def put(buf, k, x):
                for t in range(NT):
                    q_rel = t * 16 + iota16
                    d = idx_v[pl.ds(k * CQ + t * 16, 16)]
                    mask = q_rel < CQ
                    plsc.store_scatter(
                        buf,
                        [q_rel // N_COLS, lax.rem(q_rel, N_COLS), d],
                        x,
                        mask=mask,
                    )

            def start_dma(buf, sem, k):
                return pltpu.make_async_copy(
                    buf, out_ref.at[pl.ds(base_n + k * CROWS, CROWS)], sem
                )

            pltpu.make_async_copy(zc_ref, zbuf0, sem0).wait()
            pltpu.make_async_copy(zc_ref, zbuf1, sem1).wait()

            for b in range(NBUF):
                put(zbufs[b], b, ones16)
                start_dma(zbufs[b], sems[b], b).start()

            def group_body(g, carry):
                for b in range(NBUF):
                    k = g * NBUF + b
                    start_dma(zbufs[b], sems[b], k - NBUF).wait()
                    put(zbufs[b], k - NBUF, zeros16)
                    put(zbufs[b], k, ones16)
                    start_dma(zbufs[b], sems[b], k).start()
                return carry

            lax.fori_loop(1, NGROUP, group_body, 0)

            for b in range(NBUF):
                k = (NGROUP - 1) * NBUF + b
                start_dma(zbufs[b], sems[b], k).wait()

        pl.run_scoped(
            scoped,
            pltpu.VMEM((QPW + 16,), jnp.int32),
            pltpu.VMEM((CROWS, N_COLS, DEPTH), jnp.float32),
            pltpu.VMEM((CROWS, N_COLS, DEPTH), jnp.float32),
            pltpu.SemaphoreType.DMA,
            pltpu.SemaphoreType.DMA,
        )


def kernel(inputs):
    idx = inputs.astype(jnp.int32).reshape(TOT)
    idx = jnp.pad(idx, (0, 16))
    zchunk = jnp.zeros((CROWS, N_COLS, DEPTH), jnp.float32)
    init = pl.empty((N_ROWS, N_COLS, DEPTH), jnp.float32)
    _, _, out = pl.run_state(_scatter_stateful)((idx, zchunk, init))
    return out
